# baseline (speedup 1.0000x reference)
import functools

import jax
import jax.numpy as jnp
from jax import lax
from jax.experimental import pallas as pl
from jax.experimental.pallas import tpu as pltpu
from jax.experimental.pallas import tpu_sc as plsc

B = 16384
D = 256
V = 1000

_info = plsc.get_sparse_core_info()
NC = _info.num_cores
NS = _info.num_subcores
NW = NC * NS
BPW = B // NW
C = 128
NCHUNK = BPW // C

_mesh = plsc.VectorSubcoreMesh(core_axis_name="c", subcore_axis_name="s")


NBUF = 3


@functools.partial(
    pl.kernel,
    mesh=_mesh,
    out_type=jax.ShapeDtypeStruct((B, D), jnp.float32),
    compiler_params=pltpu.CompilerParams(skip_device_barrier=True),
    scratch_types=[
        pltpu.VMEM((NCHUNK, C), jnp.int32),
        pltpu.VMEM((C, D), jnp.float32),
        pltpu.VMEM((C, D), jnp.float32),
        pltpu.VMEM((C, D), jnp.float32),
        pltpu.SemaphoreType.DMA,
        pltpu.SemaphoreType.DMA,
        pltpu.SemaphoreType.DMA,
        pltpu.SemaphoreType.DMA,
        pltpu.SemaphoreType.DMA,
        pltpu.SemaphoreType.DMA,
    ],
)
def _gather_kernel(table_hbm, idx_hbm, out_hbm, idx_v,
                   b0, b1, b2, sg0, sg1, sg2, so0, so1, so2):
    wid = lax.axis_index("s") * NC + lax.axis_index("c")
    base = wid * BPW
    bufs = [b0, b1, b2]
    gsems = [sg0, sg1, sg2]
    osems = [so0, so1, so2]
    pltpu.sync_copy(idx_hbm.at[wid], idx_v)

    def gather(j):
        k = j % NBUF
        return pltpu.async_copy(table_hbm.at[idx_v.at[j]], bufs[k], gsems[k])

    def writeback(j):
        k = j % NBUF
        return pltpu.async_copy(
            bufs[k], out_hbm.at[pl.ds(base + j * C, C)], osems[k])

    gathers = [gather(j) for j in range(NBUF)]
    writes = [None] * NCHUNK
    for j in range(NCHUNK):
        gathers[j % NBUF].wait()
        writes[j] = writeback(j)
        nxt = j + NBUF
        if nxt < NCHUNK:
            writes[nxt - NBUF].wait()
            gathers[nxt % NBUF] = gather(nxt)
    for j in range(max(0, NCHUNK - NBUF), NCHUNK):
        writes[j].wait()


def kernel(emotion_ids, emb_e_weight):
    idx = emotion_ids.astype(jnp.int32).reshape(NW, NCHUNK, C)
    return _gather_kernel(emb_e_weight, idx)


# speedup vs baseline: 1.0017x; 1.0017x over previous
import functools

import jax
import jax.numpy as jnp
from jax import lax
from jax.experimental import pallas as pl
from jax.experimental.pallas import tpu as pltpu
from jax.experimental.pallas import tpu_sc as plsc

B = 16384
D = 256
V = 1000

_info = plsc.get_sparse_core_info()
NC = _info.num_cores
NS = _info.num_subcores
NW = NC * NS
BPW = B // NW
C = 128
NCHUNK = BPW // C

_mesh = plsc.VectorSubcoreMesh(core_axis_name="c", subcore_axis_name="s")


NBUF = 3


@functools.partial(
    pl.kernel,
    mesh=_mesh,
    out_type=jax.ShapeDtypeStruct((B, D), jnp.float32),
    compiler_params=pltpu.CompilerParams(use_tc_tiling_on_sc=True),
    scratch_types=[
        pltpu.VMEM((NCHUNK, C), jnp.int32),
        pltpu.VMEM((C, D), jnp.float32),
        pltpu.VMEM((C, D), jnp.float32),
        pltpu.VMEM((C, D), jnp.float32),
        pltpu.SemaphoreType.DMA,
        pltpu.SemaphoreType.DMA,
        pltpu.SemaphoreType.DMA,
        pltpu.SemaphoreType.DMA,
        pltpu.SemaphoreType.DMA,
        pltpu.SemaphoreType.DMA,
    ],
)
def _gather_kernel(table_hbm, idx_hbm, out_hbm, idx_v,
                   b0, b1, b2, sg0, sg1, sg2, so0, so1, so2):
    wid = lax.axis_index("s") * NC + lax.axis_index("c")
    base = wid * BPW
    bufs = [b0, b1, b2]
    gsems = [sg0, sg1, sg2]
    osems = [so0, so1, so2]
    pltpu.sync_copy(idx_hbm.at[wid], idx_v)

    def gather(j):
        k = j % NBUF
        return pltpu.async_copy(table_hbm.at[idx_v.at[j]], bufs[k], gsems[k])

    def writeback(j):
        k = j % NBUF
        return pltpu.async_copy(
            bufs[k], out_hbm.at[pl.ds(base + j * C, C)], osems[k])

    gathers = [gather(j) for j in range(NBUF)]
    writes = [None] * NCHUNK
    for j in range(NCHUNK):
        gathers[j % NBUF].wait()
        writes[j] = writeback(j)
        nxt = j + NBUF
        if nxt < NCHUNK:
            writes[nxt - NBUF].wait()
            gathers[nxt % NBUF] = gather(nxt)
    for j in range(max(0, NCHUNK - NBUF), NCHUNK):
        writes[j].wait()


def kernel(emotion_ids, emb_e_weight):
    idx = emotion_ids.astype(jnp.int32).reshape(NW, NCHUNK, C)
    return _gather_kernel(emb_e_weight, idx)


# speedup vs baseline: 1.0020x; 1.0003x over previous
import functools

import jax
import jax.numpy as jnp
from jax import lax
from jax.experimental import pallas as pl
from jax.experimental.pallas import tpu as pltpu
from jax.experimental.pallas import tpu_sc as plsc

B = 16384
D = 256
V = 1000

_info = plsc.get_sparse_core_info()
NC = _info.num_cores
NS = _info.num_subcores
NW = NC * NS
BPW = B // NW
C = 128
NCHUNK = BPW // C

_mesh = plsc.VectorSubcoreMesh(core_axis_name="c", subcore_axis_name="s")


NBUF = 3


@functools.partial(
    pl.kernel,
    mesh=_mesh,
    out_type=jax.ShapeDtypeStruct((B, D), jnp.float32),
    scratch_types=[
        pltpu.VMEM((NCHUNK, C), jnp.int32),
        pltpu.VMEM((C, D), jnp.float32),
        pltpu.VMEM((C, D), jnp.float32),
        pltpu.VMEM((C, D), jnp.float32),
        pltpu.SemaphoreType.DMA,
        pltpu.SemaphoreType.DMA,
        pltpu.SemaphoreType.DMA,
        pltpu.SemaphoreType.DMA,
        pltpu.SemaphoreType.DMA,
        pltpu.SemaphoreType.DMA,
    ],
)
def _gather_kernel(table_hbm, idx_hbm, out_hbm, idx_v,
                   b0, b1, b2, sg0, sg1, sg2, so0, so1, so2):
    wid = lax.axis_index("s") * NC + lax.axis_index("c")
    base = wid * BPW
    bufs = [b0, b1, b2]
    gsems = [sg0, sg1, sg2]
    osems = [so0, so1, so2]
    pltpu.sync_copy(idx_hbm.at[wid], idx_v)

    def gather(j):
        k = j % NBUF
        return pltpu.async_copy(table_hbm.at[idx_v.at[j]], bufs[k], gsems[k])

    def writeback(j):
        k = j % NBUF
        return pltpu.async_copy(
            bufs[k], out_hbm.at[pl.ds(base + j * C, C)], osems[k])

    gathers = [gather(j) for j in range(NBUF)]
    writes = [None] * NCHUNK
    for j in range(NCHUNK):
        gathers[j % NBUF].wait()
        writes[j] = writeback(j)
        nxt = j + NBUF
        if nxt < NCHUNK:
            writes[nxt - NBUF].wait()
            gathers[nxt % NBUF] = gather(nxt)
    for j in range(max(0, NCHUNK - NBUF), NCHUNK):
        writes[j].wait()


def kernel(emotion_ids, emb_e_weight):
    idx = emotion_ids.astype(jnp.int32).reshape(NW, NCHUNK, C)
    return _gather_kernel(emb_e_weight, idx)
